# baseline (speedup 1.0000x reference)
import functools

import jax
import jax.numpy as jnp
from jax import lax
from jax.experimental import pallas as pl
from jax.experimental.pallas import tpu as pltpu
from jax.experimental.pallas import tpu_sc as plsc

_NUM_EXPERTS = 64
_DROP_RATE = 0.1
_NC = 2
_NS = 16
_L = 16

_B = 4
_S = 8192
_K = 8
_ROWS_W = _S * _B // (_NC * _NS)
_CHUNK = 128
_NCHUNK = _ROWS_W // _CHUNK

_MASK_W0 = 0x77EFDFFF
_MASK_W1 = 0xFDEFFFAF


def _take16(x, idx):
    return lax.gather(
        x, idx[:, None],
        lax.GatherDimensionNumbers(
            offset_dims=(), collapsed_slice_dims=(0,), start_index_map=(0,)),
        (1,), mode=lax.GatherScatterMode.PROMISE_IN_BOUNDS)


def _sc_body(w_hbm, idx_hbm, out_hbm, oidx_hbm,
             w_v0, w_v1, idx_v0, idx_v1, out_v0, out_v1,
             sem_w0, sem_w1, sem_i0, sem_i1, sem_o0, sem_o1,
             sem_x0, sem_x1):
    wid = lax.axis_index("s") * _NC + lax.axis_index("c")
    b = wid // 8
    r0 = (wid % 8) * _ROWS_W

    w_bufs = (w_v0, w_v1)
    idx_bufs = (idx_v0, idx_v1)
    out_bufs = (out_v0, out_v1)
    sems_w = (sem_w0, sem_w1)
    sems_i = (sem_i0, sem_i1)
    sems_o = (sem_o0, sem_o1)
    sems_x = (sem_x0, sem_x1)

    lane = lax.iota(jnp.int32, _L)
    p1 = lane ^ 1
    p2 = lane ^ 2
    p4 = lane ^ 4
    rowpat = lane >> 3
    colpat = lane & 7
    w0 = jnp.full((_L,), _MASK_W0, dtype=jnp.uint32)
    w1 = jnp.full((_L,), _MASK_W1, dtype=jnp.uint32)

    cx = pltpu.async_copy(
        idx_hbm.at[b, pl.ds(r0, _ROWS_W), :],
        oidx_hbm.at[b, pl.ds(r0, _ROWS_W), :], sem_x0)

    def start_in(c):
        slot = c % 2
        rc = r0 + c * _CHUNK
        cw = pltpu.async_copy(
            w_hbm.at[b, pl.ds(rc, _CHUNK), :], w_bufs[slot], sems_w[slot])
        ci = pltpu.async_copy(
            idx_hbm.at[b, pl.ds(rc, _CHUNK), :], idx_bufs[slot], sems_i[slot])
        return cw, ci

    pending_in = {0: start_in(0)}
    pending_out = {}
    for c in range(_NCHUNK):
        slot = c % 2
        rc = r0 + c * _CHUNK
        if c + 1 < _NCHUNK:
            pending_in[c + 1] = start_in(c + 1)
        cw, ci = pending_in.pop(c)
        cw.wait()
        ci.wait()
        if c >= 2:
            pending_out.pop(c - 2).wait()

        idx_v = idx_bufs[slot]
        w_v = w_bufs[slot]
        out_v = out_bufs[slot]

        @plsc.parallel_loop(0, _CHUNK * _K // _L, unroll=8)
        def _(i):
            rows = rowpat + 2 * i
            idx = plsc.load_gather(idx_v, [rows, colpat])
            sh = (idx & 31).astype(jnp.uint32)
            bits = jnp.where(idx < 32, w0 >> sh, w1 >> sh) & 1
            m = bits.astype(jnp.float32)
            s = plsc.load_gather(w_v, [rows, colpat]) * m
            t = s + _take16(s, p1)
            t = t + _take16(t, p2)
            t = t + _take16(t, p4)
            plsc.store_scatter(out_v, [rows, colpat], s / (t + 1e-10))

        pending_out[c] = pltpu.async_copy(
            out_v, out_hbm.at[b, pl.ds(rc, _CHUNK), :], sems_o[slot])

    for c in sorted(pending_out):
        pending_out[c].wait()
    cx.wait()


@functools.partial(
    pl.kernel,
    out_type=(jax.ShapeDtypeStruct((_B, _S, _K), jnp.float32),
              jax.ShapeDtypeStruct((_B, _S, _K), jnp.int32)),
    mesh=plsc.VectorSubcoreMesh(
        core_axis_name="c", subcore_axis_name="s",
        num_cores=_NC, num_subcores=_NS),
    scratch_types=[
        pltpu.VMEM((_CHUNK, _K), jnp.float32),
        pltpu.VMEM((_CHUNK, _K), jnp.float32),
        pltpu.VMEM((_CHUNK, _K), jnp.int32),
        pltpu.VMEM((_CHUNK, _K), jnp.int32),
        pltpu.VMEM((_CHUNK, _K), jnp.float32),
        pltpu.VMEM((_CHUNK, _K), jnp.float32),
        pltpu.SemaphoreType.DMA,
        pltpu.SemaphoreType.DMA,
        pltpu.SemaphoreType.DMA,
        pltpu.SemaphoreType.DMA,
        pltpu.SemaphoreType.DMA,
        pltpu.SemaphoreType.DMA,
        pltpu.SemaphoreType.DMA,
        pltpu.SemaphoreType.DMA,
    ],
    compiler_params=pltpu.CompilerParams(
        needs_layout_passes=False, use_tc_tiling_on_sc=True),
    name="expert_dropout_sc",
)
def _expert_dropout_sc(w_hbm, idx_hbm, out_hbm, oidx_hbm,
                       w_v0, w_v1, idx_v0, idx_v1, out_v0, out_v1,
                       sem_w0, sem_w1, sem_i0, sem_i1, sem_o0, sem_o1,
                       sem_x0, sem_x1):
    _sc_body(w_hbm, idx_hbm, out_hbm, oidx_hbm,
             w_v0, w_v1, idx_v0, idx_v1, out_v0, out_v1,
             sem_w0, sem_w1, sem_i0, sem_i1, sem_o0, sem_o1,
             sem_x0, sem_x1)


def kernel(expert_weights, expert_indices):
    idx = expert_indices
    if idx.dtype != jnp.int32:
        idx = idx.astype(jnp.int32)
    out_w, out_idx = _expert_dropout_sc(expert_weights, idx)
    return (out_w, out_idx.astype(expert_indices.dtype))


# speedup vs baseline: 8.5121x; 8.5121x over previous
import functools

import jax
import jax.numpy as jnp
from jax import lax
from jax.experimental import pallas as pl
from jax.experimental.pallas import tpu as pltpu
from jax.experimental.pallas import tpu_sc as plsc

_NUM_EXPERTS = 64
_DROP_RATE = 0.1
_NC = 2
_NS = 16
_L = 16

_B = 4
_S = 8192
_K = 8
_ROWS_W = _S * _B // (_NC * _NS)
_CHUNK = 128
_NCHUNK = _ROWS_W // _CHUNK

_MASK_W0 = 0x77EFDFFF
_MASK_W1 = 0xFDEFFFAF


def _take16(x, idx):
    return lax.gather(
        x, idx[:, None],
        lax.GatherDimensionNumbers(
            offset_dims=(), collapsed_slice_dims=(0,), start_index_map=(0,)),
        (1,), mode=lax.GatherScatterMode.PROMISE_IN_BOUNDS)


def _sc_body(w_hbm, idx_hbm, out_hbm,
             w_v0, w_v1, idx_v0, idx_v1, out_v0, out_v1,
             sem_w0, sem_w1, sem_i0, sem_i1, sem_o0, sem_o1):
    wid = lax.axis_index("s") * _NC + lax.axis_index("c")
    b = wid // 8
    r0 = (wid % 8) * _ROWS_W

    w_bufs = (w_v0, w_v1)
    idx_bufs = (idx_v0, idx_v1)
    out_bufs = (out_v0, out_v1)
    sems_w = (sem_w0, sem_w1)
    sems_i = (sem_i0, sem_i1)
    sems_o = (sem_o0, sem_o1)

    lane = lax.iota(jnp.int32, _L)
    p1 = lane ^ 1
    p2 = lane ^ 2
    p4 = lane ^ 4
    rowpat = lane >> 3
    colpat = lane & 7
    w0 = jnp.full((_L,), _MASK_W0, dtype=jnp.uint32)
    w1 = jnp.full((_L,), _MASK_W1, dtype=jnp.uint32)

    def start_in(c):
        slot = c % 2
        rc = r0 + c * _CHUNK
        cw = pltpu.async_copy(
            w_hbm.at[b, pl.ds(rc, _CHUNK), :], w_bufs[slot], sems_w[slot])
        ci = pltpu.async_copy(
            idx_hbm.at[b, pl.ds(rc, _CHUNK), :], idx_bufs[slot], sems_i[slot])
        return cw, ci

    pending_in = {0: start_in(0)}
    pending_out = {}
    for c in range(_NCHUNK):
        slot = c % 2
        rc = r0 + c * _CHUNK
        if c + 1 < _NCHUNK:
            pending_in[c + 1] = start_in(c + 1)
        cw, ci = pending_in.pop(c)
        cw.wait()
        ci.wait()
        if c >= 2:
            pending_out.pop(c - 2).wait()

        idx_v = idx_bufs[slot]
        w_v = w_bufs[slot]
        out_v = out_bufs[slot]

        @plsc.parallel_loop(0, _CHUNK * _K // _L, unroll=8)
        def _(i):
            rows = rowpat + 2 * i
            idx = plsc.load_gather(idx_v, [rows, colpat])
            sh = (idx & 31).astype(jnp.uint32)
            bits = jnp.where(idx < 32, w0 >> sh, w1 >> sh) & 1
            m = bits.astype(jnp.float32)
            s = plsc.load_gather(w_v, [rows, colpat]) * m
            t = s + _take16(s, p1)
            t = t + _take16(t, p2)
            t = t + _take16(t, p4)
            plsc.store_scatter(out_v, [rows, colpat], s / (t + 1e-10))

        pending_out[c] = pltpu.async_copy(
            out_v, out_hbm.at[b, pl.ds(rc, _CHUNK), :], sems_o[slot])

    for c in sorted(pending_out):
        pending_out[c].wait()


@functools.partial(
    pl.kernel,
    out_type=jax.ShapeDtypeStruct((_B, _S, _K), jnp.float32),
    mesh=plsc.VectorSubcoreMesh(
        core_axis_name="c", subcore_axis_name="s",
        num_cores=_NC, num_subcores=_NS),
    scratch_types=[
        pltpu.VMEM((_CHUNK, _K), jnp.float32),
        pltpu.VMEM((_CHUNK, _K), jnp.float32),
        pltpu.VMEM((_CHUNK, _K), jnp.int32),
        pltpu.VMEM((_CHUNK, _K), jnp.int32),
        pltpu.VMEM((_CHUNK, _K), jnp.float32),
        pltpu.VMEM((_CHUNK, _K), jnp.float32),
        pltpu.SemaphoreType.DMA,
        pltpu.SemaphoreType.DMA,
        pltpu.SemaphoreType.DMA,
        pltpu.SemaphoreType.DMA,
        pltpu.SemaphoreType.DMA,
        pltpu.SemaphoreType.DMA,
    ],
    compiler_params=pltpu.CompilerParams(
        needs_layout_passes=False, use_tc_tiling_on_sc=True),
    name="expert_dropout_sc",
)
def _expert_dropout_sc(w_hbm, idx_hbm, out_hbm,
                       w_v0, w_v1, idx_v0, idx_v1, out_v0, out_v1,
                       sem_w0, sem_w1, sem_i0, sem_i1, sem_o0, sem_o1):
    _sc_body(w_hbm, idx_hbm, out_hbm,
             w_v0, w_v1, idx_v0, idx_v1, out_v0, out_v1,
             sem_w0, sem_w1, sem_i0, sem_i1, sem_o0, sem_o1)


def kernel(expert_weights, expert_indices):
    idx = expert_indices
    if idx.dtype != jnp.int32:
        idx = idx.astype(jnp.int32)
    out_w = _expert_dropout_sc(expert_weights, idx)
    return (out_w, expert_indices)


# speedup vs baseline: 22.6914x; 2.6658x over previous
import functools

import jax
import jax.numpy as jnp
from jax import lax
from jax.experimental import pallas as pl
from jax.experimental.pallas import tpu as pltpu
from jax.experimental.pallas import tpu_sc as plsc

_NUM_EXPERTS = 64
_DROP_RATE = 0.1
_NC = 2
_NS = 16
_L = 16

_B = 4
_S = 8192
_K = 8
_TOTAL = _B * _S * _K
_PER_W = _TOTAL // (_NC * _NS)
_GROUPS = _PER_W // (_K * 128)
_ITERS = _GROUPS * (128 // _L)

_MASK_W0 = 0x77EFDFFF
_MASK_W1 = 0xFDEFFFAF


def _sc_body(w_hbm, idx_hbm, out_hbm, w_v, idx_v, out_v, sem_w, sem_i):
    wid = lax.axis_index("s") * _NC + lax.axis_index("c")
    base = wid * _PER_W
    cp_w = pltpu.async_copy(w_hbm.at[pl.ds(base, _PER_W)], w_v, sem_w)
    cp_i = pltpu.async_copy(idx_hbm.at[pl.ds(base, _PER_W)], idx_v, sem_i)
    cp_w.wait()
    cp_i.wait()

    w0 = jnp.full((_L,), _MASK_W0, dtype=jnp.uint32)
    w1 = jnp.full((_L,), _MASK_W1, dtype=jnp.uint32)

    @plsc.parallel_loop(0, _ITERS, unroll=4)
    def _(i):
        off0 = (i >> 3) * (_K * 128) + (i & 7) * _L
        s_regs = []
        denom = None
        for k in range(_K):
            off = off0 + k * 128
            idx = idx_v[pl.ds(off, _L)]
            sh = (idx & 31).astype(jnp.uint32)
            bits = jnp.where(idx < 32, w0 >> sh, w1 >> sh) & 1
            s_k = w_v[pl.ds(off, _L)] * bits.astype(jnp.float32)
            s_regs.append(s_k)
            denom = s_k if denom is None else denom + s_k
        r = 1.0 / (denom + 1e-10)
        for k in range(_K):
            out_v[pl.ds(off0 + k * 128, _L)] = s_regs[k] * r

    pltpu.sync_copy(out_v, out_hbm.at[pl.ds(base, _PER_W)])


@functools.partial(
    pl.kernel,
    out_type=jax.ShapeDtypeStruct((_TOTAL,), jnp.float32),
    mesh=plsc.VectorSubcoreMesh(
        core_axis_name="c", subcore_axis_name="s",
        num_cores=_NC, num_subcores=_NS),
    scratch_types=[
        pltpu.VMEM((_PER_W,), jnp.float32),
        pltpu.VMEM((_PER_W,), jnp.int32),
        pltpu.VMEM((_PER_W,), jnp.float32),
        pltpu.SemaphoreType.DMA,
        pltpu.SemaphoreType.DMA,
    ],
    compiler_params=pltpu.CompilerParams(needs_layout_passes=False),
    name="expert_dropout_sc",
)
def _expert_dropout_sc(w_hbm, idx_hbm, out_hbm, w_v, idx_v, out_v,
                       sem_w, sem_i):
    _sc_body(w_hbm, idx_hbm, out_hbm, w_v, idx_v, out_v, sem_w, sem_i)


def _phys_flat(x):
    return (x.transpose(0, 2, 1).reshape(_B, _K, _S // 128, 128)
            .transpose(0, 2, 1, 3).reshape(_TOTAL))


def _unphys(flat):
    return (flat.reshape(_B, _S // 128, _K, 128).transpose(0, 2, 1, 3)
            .reshape(_B, _K, _S).transpose(0, 2, 1))


def kernel(expert_weights, expert_indices):
    idx = expert_indices
    if idx.dtype != jnp.int32:
        idx = idx.astype(jnp.int32)
    out = _expert_dropout_sc(_phys_flat(expert_weights), _phys_flat(idx))
    return (_unphys(out), expert_indices)
